# baseline (speedup 1.0000x reference)
import functools

import jax
import jax.numpy as jnp
from jax import lax
from jax.experimental import pallas as pl
from jax.experimental.pallas import tpu as pltpu
from jax.experimental.pallas import tpu_sc as plsc

N = 512
C = 256
B = 2
NC, NS = 2, 16
NW = NC * NS
RPW = N // NW

_SIZES = [B * w * w for w in (256, 128, 64, 32)]
_BASES = [0, _SIZES[0], _SIZES[0] + _SIZES[1], _SIZES[0] + _SIZES[1] + _SIZES[2]]


def _bcast(vec, i):
    return vec.at[jnp.broadcast_to(i, (16,))].get(mode="promise_in_bounds")


def _sc_body(table_h, bbox_h, bidx_h, out_h,
             bbox_v, bidx_v, idx_v, w_v, gbuf0, gbuf1, obuf, sem0, sem1):
    wid = lax.axis_index("s") * NC + lax.axis_index("c")
    roi0 = wid * RPW
    for q in range(4):
        pltpu.sync_copy(bbox_h.at[pl.ds(q * N + roi0, RPW)],
                        bbox_v.at[pl.ds(q * RPW, RPW)])
    pltpu.sync_copy(bidx_h.at[pl.ds(roi0, RPW)], bidx_v)

    x1v = bbox_v[pl.ds(0, 16)]
    y1v = bbox_v[pl.ds(16, 16)]
    x2v = bbox_v[pl.ds(32, 16)]
    y2v = bbox_v[pl.ds(48, 16)]
    bidxv = bidx_v[...]

    areav = (x2v - x1v) * (y2v - y1v)
    one, zero = jnp.int32(1), jnp.int32(0)
    lvlv = (jnp.where(areav >= 2.0 ** -7, one, zero)
            + jnp.where(areav >= 2.0 ** -5, one, zero)
            + jnp.where(areav >= 2.0 ** -3, one, zero))
    wiv = jnp.int32(256) >> lvlv
    wfv = wiv.astype(jnp.float32)
    basev = jnp.where(
        lvlv == 0, jnp.int32(_BASES[0]),
        jnp.where(lvlv == 1, jnp.int32(_BASES[1]),
                  jnp.where(lvlv == 2, jnp.int32(_BASES[2]),
                            jnp.int32(_BASES[3]))))
    rbasev = basev + bidxv * (wiv * wiv)
    x1sv = x1v * wfv
    y1sv = y1v * wfv
    roiwv = jnp.maximum(x2v * wfv - x1sv, 1.0)
    roihv = jnp.maximum(y2v * wfv - y1sv, 1.0)
    binwv = roiwv / 7.0
    binhv = roihv / 7.0

    himask = jnp.broadcast_to(jnp.int32(-65536), (16,))
    ki = lax.iota(jnp.int32, 16)
    kmask = ki < 14
    phf = (ki >> 1).astype(jnp.float32)
    iif = (ki & 1).astype(jnp.float32)

    def do_roi(r, carry):
        rbase_b = _bcast(rbasev, r)
        wi_b = _bcast(wiv, r)
        wf_b = _bcast(wfv, r)
        x1s_b = _bcast(x1sv, r)
        y1s_b = _bcast(y1sv, r)
        binw_b = _bcast(binwv, r)
        binh_b = _bcast(binhv, r)

        ys = y1s_b + phf * binh_b + (iif + 0.5) * (binh_b * 0.5)
        xs = x1s_b + phf * binw_b + (iif + 0.5) * (binw_b * 0.5)

        def prep(s):
            valid = (s >= -1.0) & (s <= wf_b) & kmask
            vf = jnp.where(valid, 0.5, 0.0)
            cc = jnp.maximum(s, 0.0)
            c0 = cc.astype(jnp.int32)
            wm1 = wi_b - 1
            at_edge = c0 >= wm1
            lo = jnp.minimum(c0, wm1)
            hi = jnp.minimum(c0 + 1, wm1)
            lof = lo.astype(jnp.float32)
            cf = jnp.where(at_edge, lof, cc)
            frac = cf - lof
            return lo, hi, frac * vf, (1.0 - frac) * vf

        ylow, yhigh, lyv, hyv = prep(ys)
        xlow, xhigh, lxv, hxv = prep(xs)
        ylo16 = rbase_b + ylow * wi_b
        yhi16 = rbase_b + yhigh * wi_b

        def build(ky, c2):
            ylo_b = _bcast(ylo16, ky)
            yhi_b = _bcast(yhi16, ky)
            hy_b = _bcast(hyv, ky)
            ly_b = _bcast(lyv, ky)
            b0 = ky * 64
            idx_v[pl.ds(b0, 16)] = ylo_b + xlow
            idx_v[pl.ds(b0 + 16, 16)] = ylo_b + xhigh
            idx_v[pl.ds(b0 + 32, 16)] = yhi_b + xlow
            idx_v[pl.ds(b0 + 48, 16)] = yhi_b + xhigh
            w_v[pl.ds(b0, 16)] = hy_b * hxv
            w_v[pl.ds(b0 + 16, 16)] = hy_b * lxv
            w_v[pl.ds(b0 + 32, 16)] = ly_b * hxv
            w_v[pl.ds(b0 + 48, 16)] = ly_b * lxv
            return c2

        lax.fori_loop(0, 14, build, 0)

        bufs = (gbuf0, gbuf1)
        sems = (sem0, sem1)
        cps = [None] * 7
        cps[0] = pltpu.async_copy(
            table_h.at[idx_v.at[pl.ds(0, 128)]], bufs[0], sems[0])
        for p in range(7):
            if p + 1 < 7:
                cps[p + 1] = pltpu.async_copy(
                    table_h.at[idx_v.at[pl.ds((p + 1) * 128, 128)]],
                    bufs[(p + 1) % 2], sems[(p + 1) % 2])
            cps[p].wait()
            gb = bufs[p % 2]

            def bin_f(pw, c3, p=p, gb=gb):
                acce = [jnp.zeros((16,), jnp.float32) for _ in range(C // 32)]
                acco = [jnp.zeros((16,), jnp.float32) for _ in range(C // 32)]
                for dky in range(2):
                    for c4 in range(4):
                        w16 = w_v[pl.ds(p * 128 + dky * 64 + c4 * 16, 16)]
                        for dkx in range(2):
                            lr = dky * 64 + c4 * 16 + (2 * pw + dkx)
                            wb = _bcast(w16, 2 * pw + dkx)
                            for v in range(C // 32):
                                vi = gb[lr, pl.ds(v * 16, 16)]
                                ev = lax.bitcast_convert_type(vi << 16, jnp.float32)
                                od = lax.bitcast_convert_type(vi & himask, jnp.float32)
                                acce[v] = acce[v] + wb * ev
                                acco[v] = acco[v] + wb * od
                orow = p * 7 + pw
                for v in range(C // 32):
                    obuf[orow, pl.ds(v * 16, 16)] = acce[v]
                    obuf[orow, pl.ds(C // 2 + v * 16, 16)] = acco[v]
                return c3

            lax.fori_loop(0, 7, bin_f, 0)
        pltpu.sync_copy(obuf, out_h.at[roi0 + r])
        return carry

    lax.fori_loop(0, RPW, do_roi, 0)


@jax.jit
def kernel(p2, p3, p4, p5, rpn_bbox, box_index):
    def _pack(p):
        u = lax.bitcast_convert_type(p, jnp.uint32)
        r = u + jnp.uint32(0x7FFF) + ((u >> 16) & jnp.uint32(1))
        lo = r[:, :C // 2] >> 16
        hi = r[:, C // 2:] & jnp.uint32(0xFFFF0000)
        return lax.bitcast_convert_type(lo | hi, jnp.int32)

    table_i = jnp.concatenate(
        [jnp.transpose(_pack(p), (0, 2, 3, 1)).reshape(-1, C // 2)
         for p in (p2, p3, p4, p5)], axis=0)
    bbox_t = rpn_bbox.T.reshape(-1)
    bidx = box_index.astype(jnp.int32)

    mesh = plsc.VectorSubcoreMesh(
        core_axis_name="c", subcore_axis_name="s",
        num_cores=NC, num_subcores=NS)
    out = pl.kernel(
        _sc_body,
        out_type=jax.ShapeDtypeStruct((N, 49, C), jnp.float32),
        mesh=mesh,
        scratch_types=[
            pltpu.VMEM((4 * RPW,), jnp.float32),
            pltpu.VMEM((RPW,), jnp.int32),
            pltpu.VMEM((896,), jnp.int32),
            pltpu.VMEM((896,), jnp.float32),
            pltpu.VMEM((128, C // 2), jnp.int32),
            pltpu.VMEM((128, C // 2), jnp.int32),
            pltpu.VMEM((49, C), jnp.float32),
            pltpu.SemaphoreType.DMA,
            pltpu.SemaphoreType.DMA,
        ],
    )(table_i, bbox_t, bidx)
    return out.reshape(N, 7, 7, C).transpose(0, 3, 1, 2)


# speedup vs baseline: 1.2609x; 1.2609x over previous
import functools

import jax
import jax.numpy as jnp
from jax import lax
from jax.experimental import pallas as pl
from jax.experimental.pallas import tpu as pltpu
from jax.experimental.pallas import tpu_sc as plsc

N = 512
C = 256
B = 2
NC, NS = 2, 16
NW = NC * NS
RPW = N // NW

_SIZES = [B * w * w for w in (256, 128, 64, 32)]
_BASES = [0, _SIZES[0], _SIZES[0] + _SIZES[1], _SIZES[0] + _SIZES[1] + _SIZES[2]]


def _bcast(vec, i):
    return vec.at[jnp.broadcast_to(i, (16,))].get(mode="promise_in_bounds")


def _sc_body(t2_h, t3_h, t4_h, t5_h, bbox_h, bidx_h, out_h,
             bbox_v, bidx_v, lvl_m, idx_v, w_v, gbuf0, gbuf1, obuf,
             sem0, sem1):
    wid = lax.axis_index("s") * NC + lax.axis_index("c")
    roi0 = wid * RPW
    for q in range(4):
        pltpu.sync_copy(bbox_h.at[pl.ds(q * N + roi0, RPW)],
                        bbox_v.at[pl.ds(q * RPW, RPW)])
    pltpu.sync_copy(bidx_h.at[pl.ds(roi0, RPW)], bidx_v)

    x1v = bbox_v[pl.ds(0, 16)]
    y1v = bbox_v[pl.ds(16, 16)]
    x2v = bbox_v[pl.ds(32, 16)]
    y2v = bbox_v[pl.ds(48, 16)]
    bidxv = bidx_v[...]

    areav = (x2v - x1v) * (y2v - y1v)
    one, zero = jnp.int32(1), jnp.int32(0)
    lvlv = (jnp.where(areav >= 2.0 ** -7, one, zero)
            + jnp.where(areav >= 2.0 ** -5, one, zero)
            + jnp.where(areav >= 2.0 ** -3, one, zero))
    wiv = jnp.int32(256) >> lvlv
    wfv = wiv.astype(jnp.float32)
    rbasev = bidxv * (wiv * wiv)
    lvl_m[pl.ds(0, 16)] = lvlv
    lvl_m[pl.ds(16, 16)] = lvlv
    x1sv = x1v * wfv
    y1sv = y1v * wfv
    roiwv = jnp.maximum(x2v * wfv - x1sv, 1.0)
    roihv = jnp.maximum(y2v * wfv - y1sv, 1.0)
    binwv = roiwv / 7.0
    binhv = roihv / 7.0

    himask = jnp.broadcast_to(jnp.int32(-65536), (16,))
    ki = lax.iota(jnp.int32, 16)
    kmask = ki < 14
    phf = (ki >> 1).astype(jnp.float32)
    iif = (ki & 1).astype(jnp.float32)

    def do_roi(r, carry):
        lvl_s = lvl_m[pl.ds(r, 16)][0]
        rbase_b = _bcast(rbasev, r)
        wi_b = _bcast(wiv, r)
        wf_b = _bcast(wfv, r)
        x1s_b = _bcast(x1sv, r)
        y1s_b = _bcast(y1sv, r)
        binw_b = _bcast(binwv, r)
        binh_b = _bcast(binhv, r)

        ys = y1s_b + phf * binh_b + (iif + 0.5) * (binh_b * 0.5)
        xs = x1s_b + phf * binw_b + (iif + 0.5) * (binw_b * 0.5)

        def prep(s):
            valid = (s >= -1.0) & (s <= wf_b) & kmask
            vf = jnp.where(valid, 0.5, 0.0)
            cc = jnp.maximum(s, 0.0)
            c0 = cc.astype(jnp.int32)
            wm1 = wi_b - 1
            at_edge = c0 >= wm1
            lo = jnp.minimum(c0, wm1)
            hi = jnp.minimum(c0 + 1, wm1)
            lof = lo.astype(jnp.float32)
            cf = jnp.where(at_edge, lof, cc)
            frac = cf - lof
            return lo, hi, frac * vf, (1.0 - frac) * vf

        ylow, yhigh, lyv, hyv = prep(ys)
        xlow, xhigh, lxv, hxv = prep(xs)
        ylo16 = rbase_b + ylow * wi_b
        yhi16 = rbase_b + yhigh * wi_b

        def build(ky, c2):
            ylo_b = _bcast(ylo16, ky)
            yhi_b = _bcast(yhi16, ky)
            hy_b = _bcast(hyv, ky)
            ly_b = _bcast(lyv, ky)
            b0 = ky * 64
            idx_v[pl.ds(b0, 16)] = ylo_b + xlow
            idx_v[pl.ds(b0 + 16, 16)] = ylo_b + xhigh
            idx_v[pl.ds(b0 + 32, 16)] = yhi_b + xlow
            idx_v[pl.ds(b0 + 48, 16)] = yhi_b + xhigh
            w_v[pl.ds(b0, 16)] = hy_b * hxv
            w_v[pl.ds(b0 + 16, 16)] = hy_b * lxv
            w_v[pl.ds(b0 + 32, 16)] = ly_b * hxv
            w_v[pl.ds(b0 + 48, 16)] = ly_b * lxv
            return c2

        lax.fori_loop(0, 14, build, 0)

        bufs = (gbuf0, gbuf1)
        sems = (sem0, sem1)

        def start_gather(pc, buf, sem):
            idxs = idx_v.at[pl.ds(pc * 128, 128)]
            for lq, th in enumerate((t2_h, t3_h, t4_h, t5_h)):
                @pl.when(lvl_s == lq)
                def _(th=th):
                    pltpu.make_async_copy(th.at[idxs], buf, sem).start()
            return pltpu.make_async_copy(t5_h.at[idxs], buf, sem)

        cps = [None] * 7
        cps[0] = start_gather(0, bufs[0], sems[0])
        for p in range(7):
            if p + 1 < 7:
                cps[p + 1] = start_gather((p + 1), bufs[(p + 1) % 2],
                                          sems[(p + 1) % 2])
            cps[p].wait()
            gb = bufs[p % 2]

            def bin_f(pw, c3, p=p, gb=gb):
                acce = [jnp.zeros((16,), jnp.float32) for _ in range(C // 32)]
                acco = [jnp.zeros((16,), jnp.float32) for _ in range(C // 32)]
                for dky in range(2):
                    for c4 in range(4):
                        w16 = w_v[pl.ds(p * 128 + dky * 64 + c4 * 16, 16)]
                        for dkx in range(2):
                            lr = dky * 64 + c4 * 16 + (2 * pw + dkx)
                            wb = _bcast(w16, 2 * pw + dkx)
                            for v in range(C // 32):
                                vi = gb[lr, pl.ds(v * 16, 16)]
                                ev = lax.bitcast_convert_type(vi << 16, jnp.float32)
                                od = lax.bitcast_convert_type(vi & himask, jnp.float32)
                                acce[v] = acce[v] + wb * ev
                                acco[v] = acco[v] + wb * od
                orow = p * 7 + pw
                for v in range(C // 32):
                    obuf[orow, pl.ds(v * 16, 16)] = acce[v]
                    obuf[orow, pl.ds(C // 2 + v * 16, 16)] = acco[v]
                return c3

            lax.fori_loop(0, 7, bin_f, 0)
        pltpu.sync_copy(obuf, out_h.at[roi0 + r])
        return carry

    lax.fori_loop(0, RPW, do_roi, 0)


@jax.jit
def kernel(p2, p3, p4, p5, rpn_bbox, box_index):
    def _pack(p):
        u = lax.bitcast_convert_type(p, jnp.uint32)
        r = u + jnp.uint32(0x7FFF) + ((u >> 16) & jnp.uint32(1))
        lo = r[:, :C // 2] >> 16
        hi = r[:, C // 2:] & jnp.uint32(0xFFFF0000)
        return lax.bitcast_convert_type(lo | hi, jnp.int32)

    tables = [jnp.transpose(_pack(p), (0, 2, 3, 1)).reshape(-1, C // 2)
              for p in (p2, p3, p4, p5)]
    bbox_t = rpn_bbox.T.reshape(-1)
    bidx = box_index.astype(jnp.int32)

    mesh = plsc.VectorSubcoreMesh(
        core_axis_name="c", subcore_axis_name="s",
        num_cores=NC, num_subcores=NS)
    out = pl.kernel(
        _sc_body,
        out_type=jax.ShapeDtypeStruct((N, 49, C), jnp.float32),
        mesh=mesh,
        scratch_types=[
            pltpu.VMEM((4 * RPW,), jnp.float32),
            pltpu.VMEM((RPW,), jnp.int32),
            pltpu.VMEM((32,), jnp.int32),
            pltpu.VMEM((896,), jnp.int32),
            pltpu.VMEM((896,), jnp.float32),
            pltpu.VMEM((128, C // 2), jnp.int32),
            pltpu.VMEM((128, C // 2), jnp.int32),
            pltpu.VMEM((49, C), jnp.float32),
            pltpu.SemaphoreType.DMA,
            pltpu.SemaphoreType.DMA,
        ],
    )(*tables, bbox_t, bidx)
    return out.reshape(N, 7, 7, C).transpose(0, 3, 1, 2)


# speedup vs baseline: 1.2647x; 1.0030x over previous
import functools

import jax
import jax.numpy as jnp
from jax import lax
from jax.experimental import pallas as pl
from jax.experimental.pallas import tpu as pltpu
from jax.experimental.pallas import tpu_sc as plsc

N = 512
C = 256
B = 2
NC, NS = 2, 16
NW = NC * NS
RPW = N // NW

_SIZES = [B * w * w for w in (256, 128, 64, 32)]
_BASES = [0, _SIZES[0], _SIZES[0] + _SIZES[1], _SIZES[0] + _SIZES[1] + _SIZES[2]]


def _bcast(vec, i):
    return vec.at[jnp.broadcast_to(i, (16,))].get(mode="promise_in_bounds")


def _sc_body(t2_h, t3_h, t4_h, t5_h, bbox_h, bidx_h, out_h,
             bbox_v, bidx_v, lvl_m, idx_v, w_v, gbuf0, gbuf1, obuf,
             sem0, sem1):
    wid = lax.axis_index("s") * NC + lax.axis_index("c")
    roi0 = wid * RPW
    for q in range(4):
        pltpu.sync_copy(bbox_h.at[pl.ds(q * N + roi0, RPW)],
                        bbox_v.at[pl.ds(q * RPW, RPW)])
    pltpu.sync_copy(bidx_h.at[pl.ds(roi0, RPW)], bidx_v)

    x1v = bbox_v[pl.ds(0, 16)]
    y1v = bbox_v[pl.ds(16, 16)]
    x2v = bbox_v[pl.ds(32, 16)]
    y2v = bbox_v[pl.ds(48, 16)]
    bidxv = bidx_v[...]

    areav = (x2v - x1v) * (y2v - y1v)
    one, zero = jnp.int32(1), jnp.int32(0)
    lvlv = (jnp.where(areav >= 2.0 ** -7, one, zero)
            + jnp.where(areav >= 2.0 ** -5, one, zero)
            + jnp.where(areav >= 2.0 ** -3, one, zero))
    wiv = jnp.int32(256) >> lvlv
    wfv = wiv.astype(jnp.float32)
    rbasev = bidxv * (wiv * wiv)
    lvl_m[pl.ds(0, 16)] = lvlv
    lvl_m[pl.ds(16, 16)] = lvlv
    x1sv = x1v * wfv
    y1sv = y1v * wfv
    roiwv = jnp.maximum(x2v * wfv - x1sv, 1.0)
    roihv = jnp.maximum(y2v * wfv - y1sv, 1.0)
    binwv = roiwv / 7.0
    binhv = roihv / 7.0

    himask = jnp.broadcast_to(jnp.int32(-65536), (16,))
    ki = lax.iota(jnp.int32, 16)
    kmask = ki < 14
    phf = (ki >> 1).astype(jnp.float32)
    iif = (ki & 1).astype(jnp.float32)

    def do_roi(r, carry):
        lvl_s = lvl_m[pl.ds(r, 16)][0]
        rbase_b = _bcast(rbasev, r)
        wi_b = _bcast(wiv, r)
        wf_b = _bcast(wfv, r)
        x1s_b = _bcast(x1sv, r)
        y1s_b = _bcast(y1sv, r)
        binw_b = _bcast(binwv, r)
        binh_b = _bcast(binhv, r)

        ys = y1s_b + phf * binh_b + (iif + 0.5) * (binh_b * 0.5)
        xs = x1s_b + phf * binw_b + (iif + 0.5) * (binw_b * 0.5)

        def prep(s):
            valid = (s >= -1.0) & (s <= wf_b) & kmask
            vf = jnp.where(valid, 0.5, 0.0)
            cc = jnp.maximum(s, 0.0)
            c0 = cc.astype(jnp.int32)
            wm1 = wi_b - 1
            at_edge = c0 >= wm1
            lo = jnp.minimum(c0, wm1)
            hi = jnp.minimum(c0 + 1, wm1)
            lof = lo.astype(jnp.float32)
            cf = jnp.where(at_edge, lof, cc)
            frac = cf - lof
            return lo, hi, frac * vf, (1.0 - frac) * vf

        ylow, yhigh, lyv, hyv = prep(ys)
        xlow, xhigh, lxv, hxv = prep(xs)
        ylo16 = rbase_b + ylow * wi_b
        yhi16 = rbase_b + yhigh * wi_b

        def build(ky, c2):
            ylo_b = _bcast(ylo16, ky)
            yhi_b = _bcast(yhi16, ky)
            hy_b = _bcast(hyv, ky)
            ly_b = _bcast(lyv, ky)
            b0 = ky * 64
            idx_v[pl.ds(b0, 16)] = ylo_b + xlow
            idx_v[pl.ds(b0 + 16, 16)] = ylo_b + xhigh
            idx_v[pl.ds(b0 + 32, 16)] = yhi_b + xlow
            idx_v[pl.ds(b0 + 48, 16)] = yhi_b + xhigh
            w_v[pl.ds(b0, 16)] = hy_b * hxv
            w_v[pl.ds(b0 + 16, 16)] = hy_b * lxv
            w_v[pl.ds(b0 + 32, 16)] = ly_b * hxv
            w_v[pl.ds(b0 + 48, 16)] = ly_b * lxv
            return c2

        lax.fori_loop(0, 14, build, 0)

        bufs = (gbuf0, gbuf1)
        sems = (sem0, sem1)

        def start_gather(pc, buf, sem):
            idxs = idx_v.at[pl.ds(pc * 128, 128)]
            for lq, th in enumerate((t2_h, t3_h, t4_h, t5_h)):
                @pl.when(lvl_s == lq)
                def _(th=th):
                    pltpu.make_async_copy(th.at[idxs], buf, sem).start()
            return pltpu.make_async_copy(t5_h.at[idxs], buf, sem)

        cps = [None] * 7
        cps[0] = start_gather(0, bufs[0], sems[0])
        for p in range(7):
            if p + 1 < 7:
                cps[p + 1] = start_gather((p + 1), bufs[(p + 1) % 2],
                                          sems[(p + 1) % 2])
            cps[p].wait()
            gb = bufs[p % 2]

            def bin_f(pw, c3, p=p, gb=gb):
                acce = [jnp.zeros((16,), jnp.float32) for _ in range(C // 32)]
                acco = [jnp.zeros((16,), jnp.float32) for _ in range(C // 32)]
                for dky in range(2):
                    for c4 in range(4):
                        w16 = w_v[pl.ds(p * 128 + dky * 64 + c4 * 16, 16)]
                        for dkx in range(2):
                            lr = dky * 64 + c4 * 16 + (2 * pw + dkx)
                            wb = _bcast(w16, 2 * pw + dkx)
                            for v in range(C // 32):
                                vi = gb[lr, pl.ds(v * 16, 16)]
                                ev = lax.bitcast_convert_type(vi << 16, jnp.float32)
                                od = lax.bitcast_convert_type(vi & himask, jnp.float32)
                                acce[v] = acce[v] + wb * ev
                                acco[v] = acco[v] + wb * od
                orow = p * 7 + pw
                for v in range(C // 32):
                    obuf[orow, pl.ds(v * 16, 16)] = acce[v]
                    obuf[orow, pl.ds(C // 2 + v * 16, 16)] = acco[v]
                return c3

            lax.fori_loop(0, 7, bin_f, 0)
        pltpu.sync_copy(obuf, out_h.at[roi0 + r])
        return carry

    lax.fori_loop(0, RPW, do_roi, 0)


@jax.jit
def kernel(p2, p3, p4, p5, rpn_bbox, box_index):
    def _rnd(v):
        return v + jnp.uint32(0x7FFF) + ((v >> 16) & jnp.uint32(1))

    def _pack(p):
        u = lax.bitcast_convert_type(p, jnp.uint32)
        lo = _rnd(u[:, :C // 2]) >> 16
        hi = _rnd(u[:, C // 2:]) & jnp.uint32(0xFFFF0000)
        return lax.bitcast_convert_type(lo | hi, jnp.int32)

    tables = [jnp.transpose(_pack(p), (0, 2, 3, 1)).reshape(-1, C // 2)
              for p in (p2, p3, p4, p5)]
    bbox_t = rpn_bbox.T.reshape(-1)
    bidx = box_index.astype(jnp.int32)

    mesh = plsc.VectorSubcoreMesh(
        core_axis_name="c", subcore_axis_name="s",
        num_cores=NC, num_subcores=NS)
    out = pl.kernel(
        _sc_body,
        out_type=jax.ShapeDtypeStruct((N, 49, C), jnp.float32),
        mesh=mesh,
        scratch_types=[
            pltpu.VMEM((4 * RPW,), jnp.float32),
            pltpu.VMEM((RPW,), jnp.int32),
            pltpu.VMEM((32,), jnp.int32),
            pltpu.VMEM((896,), jnp.int32),
            pltpu.VMEM((896,), jnp.float32),
            pltpu.VMEM((128, C // 2), jnp.int32),
            pltpu.VMEM((128, C // 2), jnp.int32),
            pltpu.VMEM((49, C), jnp.float32),
            pltpu.SemaphoreType.DMA,
            pltpu.SemaphoreType.DMA,
        ],
    )(*tables, bbox_t, bidx)
    return out.reshape(N, 7, 7, C).transpose(0, 3, 1, 2)
